# baseline (speedup 1.0000x reference)
import functools

import jax
import jax.numpy as jnp
from jax import lax
from jax.experimental import pallas as pl
from jax.experimental.pallas import tpu as pltpu
from jax.experimental.pallas import tpu_sc as plsc

NUM_CORES = 2
NUM_SUBCORES = 16
LANES = 16
NW = NUM_CORES * NUM_SUBCORES

BATCH = 16384
FEAT_DIM = 32
BPW = BATCH // NW
TILE_W = 128
NBUF = 20
PH = 8
NPH = BPW // PH
SLAB = BATCH // NUM_SUBCORES
FWORDS = BATCH * FEAT_DIM


def _partial_sums(features_t, labels_s, perm, centers_t):
    mesh = plsc.VectorSubcoreMesh(core_axis_name="c", subcore_axis_name="s")

    @functools.partial(
        pl.kernel,
        mesh=mesh,
        out_type=(
            jax.ShapeDtypeStruct((NW * LANES,), jnp.float32),
            jax.ShapeDtypeStruct((FWORDS,), jnp.float32),
            jax.ShapeDtypeStruct((FWORDS,), jnp.float32),
        ),
        compiler_params=pltpu.CompilerParams(use_tc_tiling_on_sc=True,
                                             needs_layout_passes=False),
        scratch_types=[
            pltpu.VMEM((BPW + LANES,), jnp.int32),
            pltpu.VMEM((BPW,), jnp.int32),
            pltpu.VMEM((NBUF, FEAT_DIM, TILE_W), jnp.float32),
            pltpu.VMEM((FWORDS // NW,), jnp.float32),
            pltpu.VMEM((TILE_W, TILE_W), jnp.int32),
            pltpu.VMEM((FEAT_DIM, TILE_W), jnp.float32),
            pltpu.VMEM((TILE_W * FEAT_DIM,), jnp.float32),
            pltpu.VMEM((LANES,), jnp.float32),
            pltpu.SMEM((4, PH), jnp.int32),
            pltpu.SemaphoreType.DMA,
            pltpu.SemaphoreType.DMA,
        ],
    )
    def k(feat_hbm, lab_hbm, perm_hbm, cent_hbm,
          out_hbm, fsc0, fsc1,
          idx_v, perm_v, cbuf, fstage, idxf, tin, tout, acc_v, slot_sm,
          sem, fsem):
        cid = lax.axis_index("c")
        sid = lax.axis_index("s")
        wid = sid * NUM_CORES + cid
        base = wid * BPW
        iota = lax.iota(jnp.int32, LANES)
        rows_lo = iota
        rows_hi = iota + LANES

        def stage(ch, carry):
            off = sid * SLAB + ch * TILE_W
            pltpu.sync_copy(feat_hbm.at[:, pl.ds(off, TILE_W)], tin)
            for c16 in range(TILE_W // LANES):
                rows = jnp.full((LANES,), c16 * LANES, jnp.int32) + iota
                for d in range(FEAT_DIM):
                    pos = rows * FEAT_DIM + d
                    plsc.store_scatter(tout, [pos],
                                       tin[d, pl.ds(c16 * LANES, LANES)])

            @pl.when(cid == 0)
            def _():
                pltpu.sync_copy(tout, fsc0.at[pl.ds(off * FEAT_DIM,
                                                    TILE_W * FEAT_DIM)])

            @pl.when(cid == 1)
            def _():
                pltpu.sync_copy(tout, fsc1.at[pl.ds(off * FEAT_DIM,
                                                    TILE_W * FEAT_DIM)])
            return carry

        lax.fori_loop(0, SLAB // TILE_W, stage, 0)
        plsc.subcore_barrier()

        pltpu.sync_copy(lab_hbm.at[pl.ds(base, BPW)], idx_v.at[pl.ds(0, BPW)])
        pltpu.sync_copy(perm_hbm.at[pl.ds(base, BPW)], perm_v)

        def fidx(g, carry):
            pv = perm_v[pl.ds(g * LANES, LANES)] * FEAT_DIM
            pos0 = (jnp.full((LANES,), g * LANES, jnp.int32) + iota) * FEAT_DIM
            for d in range(FEAT_DIM):
                pos = pos0 + d
                plsc.store_scatter(idxf, [pos >> 7, pos & (TILE_W - 1)],
                                   pv + d)
            return carry

        lax.fori_loop(0, BPW // LANES, fidx, 0)

        def ffire(j, carry):
            @pl.when(cid == 0)
            def _():
                pltpu.async_copy(fsc0.at[idxf.at[j]],
                                 fstage.at[pl.ds(j * TILE_W, TILE_W)], fsem)

            @pl.when(cid == 1)
            def _():
                pltpu.async_copy(fsc1.at[idxf.at[j]],
                                 fstage.at[pl.ds(j * TILE_W, TILE_W)], fsem)
            return carry

        lax.fori_loop(0, BPW * FEAT_DIM // TILE_W, ffire, 0)
        pltpu.make_async_copy(fsc0.at[pl.ds(0, FWORDS // NW)], fstage,
                              fsem).wait()

        def fire_phase(q, nf, cur_col):
            vec = idx_v[pl.ds(q * PH, LANES)]
            nf0 = nf
            for i in range(PH):
                col = pl.multiple_of((vec[i] >> 7) << 7, TILE_W)
                neq = col != cur_col
                slot = lax.rem(nf, NBUF)

                @pl.when(neq)
                def _():
                    pltpu.async_copy(cent_hbm.at[:, pl.ds(col, TILE_W)],
                                     cbuf.at[slot], sem)

                nf = jnp.where(neq, nf + 1, nf)
                cur_col = jnp.where(neq, col, cur_col)
                slot_sm[lax.rem(q, 4), i] = lax.rem(nf - 1, NBUF)
            return nf, cur_col, nf - nf0

        def waits(n):
            def wbody(j, c):
                pltpu.make_async_copy(cent_hbm.at[:, pl.ds(0, TILE_W)],
                                      cbuf.at[0], sem).wait()
                return c
            lax.fori_loop(0, n, wbody, 0)

        def consume_phase(p, acc):
            vec = idx_v[pl.ds(p * PH, LANES)]
            for i in range(PH):
                lane = jnp.full((LANES,), vec[i] & (TILE_W - 1), jnp.int32)
                slot = jnp.full((LANES,), slot_sm[lax.rem(p, 4), i], jnp.int32)
                c_lo = plsc.load_gather(cbuf, [slot, rows_lo, lane])
                c_hi = plsc.load_gather(cbuf, [slot, rows_hi, lane])
                fb = (p * PH + i) * FEAT_DIM
                f_lo = fstage[pl.ds(fb, LANES)]
                f_hi = fstage[pl.ds(fb + LANES, LANES)]
                d_lo = f_lo - c_lo
                d_hi = f_hi - c_hi
                acc = acc + d_lo * d_lo + d_hi * d_hi
            return acc

        nf, cur_col, cnt0 = fire_phase(0, jnp.int32(0), jnp.int32(-1))
        nf, cur_col, cnt1 = fire_phase(1, nf, cur_col)

        def body(p, carry):
            cur_col, nf, cnt_a, cnt_b, acc = carry
            waits(cnt_a)
            acc = consume_phase(p, acc)
            nf, cur_col, cnt_new = fire_phase(p + 2, nf, cur_col)
            return cur_col, nf, cnt_b, cnt_new, acc

        cur_col, nf, cnt_a, cnt_b, acc = lax.fori_loop(
            0, NPH - 2, body,
            (cur_col, nf, cnt0, cnt1, jnp.zeros((LANES,), jnp.float32)))

        waits(cnt_a)
        acc = consume_phase(NPH - 2, acc)
        waits(cnt_b)
        acc = consume_phase(NPH - 1, acc)

        acc_v[...] = acc
        pltpu.sync_copy(acc_v, out_hbm.at[pl.ds(wid * LANES, LANES)])

    return k(features_t, labels_s, perm, centers_t)


def kernel(features, labels, centers):
    labels32 = labels.astype(jnp.int32)
    perm = jnp.argsort(labels32).astype(jnp.int32)
    labels_s = jnp.take(labels32, perm)
    partials, _, _ = _partial_sums(features.T, labels_s, perm, centers.T)
    return jnp.sum(partials) * (0.5 / BATCH)


# speedup vs baseline: 1.0118x; 1.0118x over previous
import functools

import jax
import jax.numpy as jnp
from jax import lax
from jax.experimental import pallas as pl
from jax.experimental.pallas import tpu as pltpu
from jax.experimental.pallas import tpu_sc as plsc

NUM_CORES = 2
NUM_SUBCORES = 16
LANES = 16
NW = NUM_CORES * NUM_SUBCORES

BATCH = 16384
FEAT_DIM = 32
BPW = BATCH // NW
TILE_W = 128
NBUF = 20
PH = 8
NPH = BPW // PH
SLAB = BATCH // NUM_SUBCORES
FWORDS = BATCH * FEAT_DIM


def _partial_sums(features_t, labels_s, perm, centers_t):
    mesh = plsc.VectorSubcoreMesh(core_axis_name="c", subcore_axis_name="s")

    @functools.partial(
        pl.kernel,
        mesh=mesh,
        out_type=(
            jax.ShapeDtypeStruct((NW * LANES,), jnp.float32),
            jax.ShapeDtypeStruct((FWORDS,), jnp.float32),
            jax.ShapeDtypeStruct((FWORDS,), jnp.float32),
        ),
        compiler_params=pltpu.CompilerParams(use_tc_tiling_on_sc=True,
                                             needs_layout_passes=False),
        scratch_types=[
            pltpu.VMEM((BPW + LANES,), jnp.int32),
            pltpu.VMEM((BPW,), jnp.int32),
            pltpu.VMEM((NBUF, FEAT_DIM, TILE_W), jnp.float32),
            pltpu.VMEM((FWORDS // NW,), jnp.float32),
            pltpu.VMEM((TILE_W, TILE_W), jnp.int32),
            pltpu.VMEM((FEAT_DIM, TILE_W), jnp.float32),
            pltpu.VMEM((TILE_W * FEAT_DIM,), jnp.float32),
            pltpu.VMEM((LANES,), jnp.float32),
            pltpu.SMEM((4, PH), jnp.int32),
            pltpu.SemaphoreType.DMA,
            pltpu.SemaphoreType.DMA,
        ],
    )
    def k(feat_hbm, lab_hbm, perm_hbm, cent_hbm,
          out_hbm, fsc0, fsc1,
          idx_v, perm_v, cbuf, fstage, idxf, tin, tout, acc_v, slot_sm,
          sem, fsem):
        cid = lax.axis_index("c")
        sid = lax.axis_index("s")
        wid = sid * NUM_CORES + cid
        base = wid * BPW
        iota = lax.iota(jnp.int32, LANES)
        rows_lo = iota
        rows_hi = iota + LANES

        def fire_phase(q, nf, cur_col):
            vec = idx_v[pl.ds(q * PH, LANES)]
            nf0 = nf
            for i in range(PH):
                col = pl.multiple_of((vec[i] >> 7) << 7, TILE_W)
                neq = col != cur_col
                slot = lax.rem(nf, NBUF)

                @pl.when(neq)
                def _():
                    pltpu.async_copy(cent_hbm.at[:, pl.ds(col, TILE_W)],
                                     cbuf.at[slot], sem)

                nf = jnp.where(neq, nf + 1, nf)
                cur_col = jnp.where(neq, col, cur_col)
                slot_sm[lax.rem(q, 4), i] = lax.rem(nf - 1, NBUF)
            return nf, cur_col, nf - nf0

        pltpu.sync_copy(lab_hbm.at[pl.ds(base, BPW)], idx_v.at[pl.ds(0, BPW)])
        pltpu.sync_copy(perm_hbm.at[pl.ds(base, BPW)], perm_v)
        nf, cur_col, cnt0 = fire_phase(0, jnp.int32(0), jnp.int32(-1))
        nf, cur_col, cnt1 = fire_phase(1, nf, cur_col)

        def stage(ch, carry):
            off = sid * SLAB + ch * TILE_W
            pltpu.sync_copy(feat_hbm.at[:, pl.ds(off, TILE_W)], tin)
            for c16 in range(TILE_W // LANES):
                rows = jnp.full((LANES,), c16 * LANES, jnp.int32) + iota
                for d in range(FEAT_DIM):
                    pos = rows * FEAT_DIM + d
                    plsc.store_scatter(tout, [pos],
                                       tin[d, pl.ds(c16 * LANES, LANES)])

            @pl.when(cid == 0)
            def _():
                pltpu.sync_copy(tout, fsc0.at[pl.ds(off * FEAT_DIM,
                                                    TILE_W * FEAT_DIM)])

            @pl.when(cid == 1)
            def _():
                pltpu.sync_copy(tout, fsc1.at[pl.ds(off * FEAT_DIM,
                                                    TILE_W * FEAT_DIM)])
            return carry

        lax.fori_loop(0, SLAB // TILE_W, stage, 0)

        def fidx(g, carry):
            pv = perm_v[pl.ds(g * LANES, LANES)] * FEAT_DIM
            pos0 = (jnp.full((LANES,), g * LANES, jnp.int32) + iota) * FEAT_DIM
            for d in range(FEAT_DIM):
                pos = pos0 + d
                plsc.store_scatter(idxf, [pos >> 7, pos & (TILE_W - 1)],
                                   pv + d)
            return carry

        lax.fori_loop(0, BPW // LANES, fidx, 0)
        plsc.subcore_barrier()

        def ffire(j, carry):
            @pl.when(cid == 0)
            def _():
                pltpu.async_copy(fsc0.at[idxf.at[j]],
                                 fstage.at[pl.ds(j * TILE_W, TILE_W)], fsem)

            @pl.when(cid == 1)
            def _():
                pltpu.async_copy(fsc1.at[idxf.at[j]],
                                 fstage.at[pl.ds(j * TILE_W, TILE_W)], fsem)
            return carry

        lax.fori_loop(0, BPW * FEAT_DIM // LANES // PH, ffire, 0)

        def fwait(p):
            pltpu.make_async_copy(
                fsc0.at[pl.ds(0, PH * FEAT_DIM)],
                fstage.at[pl.ds(p * PH * FEAT_DIM, PH * FEAT_DIM)],
                fsem).wait()

        def waits(n):
            def wbody(j, c):
                pltpu.make_async_copy(cent_hbm.at[:, pl.ds(0, TILE_W)],
                                      cbuf.at[0], sem).wait()
                return c
            lax.fori_loop(0, n, wbody, 0)

        def consume_phase(p, acc):
            vec = idx_v[pl.ds(p * PH, LANES)]
            for i in range(PH):
                lane = jnp.full((LANES,), vec[i] & (TILE_W - 1), jnp.int32)
                slot = jnp.full((LANES,), slot_sm[lax.rem(p, 4), i], jnp.int32)
                c_lo = plsc.load_gather(cbuf, [slot, rows_lo, lane])
                c_hi = plsc.load_gather(cbuf, [slot, rows_hi, lane])
                fb = (p * PH + i) * FEAT_DIM
                f_lo = fstage[pl.ds(fb, LANES)]
                f_hi = fstage[pl.ds(fb + LANES, LANES)]
                d_lo = f_lo - c_lo
                d_hi = f_hi - c_hi
                acc = acc + d_lo * d_lo + d_hi * d_hi
            return acc

        def body(p, carry):
            cur_col, nf, cnt_a, cnt_b, acc = carry
            waits(cnt_a)
            fwait(p)
            acc = consume_phase(p, acc)
            nf, cur_col, cnt_new = fire_phase(p + 2, nf, cur_col)
            return cur_col, nf, cnt_b, cnt_new, acc

        cur_col, nf, cnt_a, cnt_b, acc = lax.fori_loop(
            0, NPH - 2, body,
            (cur_col, nf, cnt0, cnt1, jnp.zeros((LANES,), jnp.float32)))

        waits(cnt_a)
        fwait(NPH - 2)
        acc = consume_phase(NPH - 2, acc)
        waits(cnt_b)
        fwait(NPH - 1)
        acc = consume_phase(NPH - 1, acc)

        acc_v[...] = acc
        pltpu.sync_copy(acc_v, out_hbm.at[pl.ds(wid * LANES, LANES)])

    return k(features_t, labels_s, perm, centers_t)


def kernel(features, labels, centers):
    labels32 = labels.astype(jnp.int32)
    perm = jnp.argsort(labels32).astype(jnp.int32)
    labels_s = jnp.take(labels32, perm)
    partials, _, _ = _partial_sums(features.T, labels_s, perm, centers.T)
    return jnp.sum(partials) * (0.5 / BATCH)


# speedup vs baseline: 1.3732x; 1.3573x over previous
import functools

import jax
import jax.numpy as jnp
from jax import lax
from jax.experimental import pallas as pl
from jax.experimental.pallas import tpu as pltpu
from jax.experimental.pallas import tpu_sc as plsc

NUM_CORES = 2
NUM_SUBCORES = 16
LANES = 16
NW = NUM_CORES * NUM_SUBCORES

BATCH = 16384
FEAT_DIM = 32
BPW = BATCH // NW
TILE_W = 128
NBUF = 20
PH = 8
NPH = BPW // PH


def _partial_sums(features_t, labels_s, centers_t):
    mesh = plsc.VectorSubcoreMesh(core_axis_name="c", subcore_axis_name="s")

    @functools.partial(
        pl.kernel,
        mesh=mesh,
        out_type=jax.ShapeDtypeStruct((NW * LANES,), jnp.float32),
        compiler_params=pltpu.CompilerParams(use_tc_tiling_on_sc=True,
                                             needs_layout_passes=False),
        scratch_types=[
            pltpu.VMEM((BPW + LANES,), jnp.int32),
            pltpu.VMEM((FEAT_DIM, BPW), jnp.float32),
            pltpu.VMEM((NBUF, FEAT_DIM, TILE_W), jnp.float32),
            pltpu.VMEM((LANES,), jnp.float32),
            pltpu.SMEM((4, PH), jnp.int32),
            pltpu.SemaphoreType.DMA,
            pltpu.SemaphoreType.DMA,
        ],
    )
    def k(feat_hbm, lab_hbm, cent_hbm, out_hbm,
          idx_v, feat_v, cbuf, acc_v, slot_sm, sem, fsem):
        cid = lax.axis_index("c")
        sid = lax.axis_index("s")
        wid = sid * NUM_CORES + cid
        base = wid * BPW
        iota = lax.iota(jnp.int32, LANES)
        rows_lo = iota
        rows_hi = iota + LANES

        pltpu.sync_copy(lab_hbm.at[pl.ds(base, BPW)], idx_v.at[pl.ds(0, BPW)])
        feat_cp = pltpu.async_copy(feat_hbm.at[:, pl.ds(base, BPW)],
                                   feat_v, fsem)

        def fire_phase(q, nf, cur_col):
            vec = idx_v[pl.ds(q * PH, LANES)]
            nf0 = nf
            for i in range(PH):
                col = pl.multiple_of((vec[i] >> 7) << 7, TILE_W)
                neq = col != cur_col
                slot = lax.rem(nf, NBUF)

                @pl.when(neq)
                def _():
                    pltpu.async_copy(cent_hbm.at[:, pl.ds(col, TILE_W)],
                                     cbuf.at[slot], sem)

                nf = jnp.where(neq, nf + 1, nf)
                cur_col = jnp.where(neq, col, cur_col)
                slot_sm[lax.rem(q, 4), i] = lax.rem(nf - 1, NBUF)
            return nf, cur_col, nf - nf0

        def waits(n):
            def wbody(j, c):
                pltpu.make_async_copy(cent_hbm.at[:, pl.ds(0, TILE_W)],
                                      cbuf.at[0], sem).wait()
                return c
            lax.fori_loop(0, n, wbody, 0)

        def consume_phase(p, acc):
            vec = idx_v[pl.ds(p * PH, LANES)]
            for i in range(PH):
                lane = jnp.full((LANES,), vec[i] & (TILE_W - 1), jnp.int32)
                slot = jnp.full((LANES,), slot_sm[lax.rem(p, 4), i], jnp.int32)
                item = jnp.full((LANES,), p * PH + i, jnp.int32)
                c_lo = plsc.load_gather(cbuf, [slot, rows_lo, lane])
                c_hi = plsc.load_gather(cbuf, [slot, rows_hi, lane])
                f_lo = plsc.load_gather(feat_v, [rows_lo, item])
                f_hi = plsc.load_gather(feat_v, [rows_hi, item])
                d_lo = f_lo - c_lo
                d_hi = f_hi - c_hi
                acc = acc + d_lo * d_lo + d_hi * d_hi
            return acc

        nf, cur_col, cnt0 = fire_phase(0, jnp.int32(0), jnp.int32(-1))
        nf, cur_col, cnt1 = fire_phase(1, nf, cur_col)
        feat_cp.wait()

        def body(p, carry):
            cur_col, nf, cnt_a, cnt_b, acc = carry
            waits(cnt_a)
            acc = consume_phase(p, acc)
            nf, cur_col, cnt_new = fire_phase(p + 2, nf, cur_col)
            return cur_col, nf, cnt_b, cnt_new, acc

        cur_col, nf, cnt_a, cnt_b, acc = lax.fori_loop(
            0, NPH - 2, body,
            (cur_col, nf, cnt0, cnt1, jnp.zeros((LANES,), jnp.float32)))

        waits(cnt_a)
        acc = consume_phase(NPH - 2, acc)
        waits(cnt_b)
        acc = consume_phase(NPH - 1, acc)

        acc_v[...] = acc
        pltpu.sync_copy(acc_v, out_hbm.at[pl.ds(wid * LANES, LANES)])

    return k(features_t, labels_s, centers_t)


def kernel(features, labels, centers):
    labels32 = labels.astype(jnp.int32)
    perm = jnp.argsort(labels32)
    labels_s = jnp.take(labels32, perm)
    features_s = jnp.take(features, perm, axis=0)
    partials = _partial_sums(features_s.T, labels_s, centers.T)
    return jnp.sum(partials) * (0.5 / BATCH)
